# Initial kernel scaffold; baseline (speedup 1.0000x reference)
import jax
import jax.numpy as jnp
from jax.experimental import pallas as pl


def kernel(goods_ids, shop_ids, cate_ids, goods_prices, goods_table, shop_table, cate_table, price_table):
    raise NotImplementedError("write your pallas kernel here")


# speedup vs baseline: 9.4314x; 9.4314x over previous
import functools

import jax
import jax.numpy as jnp
from jax import lax
from jax.experimental import pallas as pl
from jax.experimental.pallas import tpu as pltpu
from jax.experimental.pallas import tpu_sc as plsc

B, L, D = 4096, 200, 32
N = B * L
NF = 4

NC, NS = 2, 16
NW = NC * NS
PER_W = N // NW

IDROWS = 8
SUPER = IDROWS * 128
N_SUPER = PER_W // SUPER

_mesh = plsc.VectorSubcoreMesh(core_axis_name="c", subcore_axis_name="s")


@functools.partial(
    pl.kernel,
    mesh=_mesh,
    out_type=jax.ShapeDtypeStruct((N, NF * D), jnp.float32),
    scratch_types=[
        pltpu.VMEM((NF, IDROWS, 128), jnp.int32),
        pltpu.VMEM((SUPER, D), jnp.float32),
        pltpu.SemaphoreType.DMA,
    ],
    compiler_params=pltpu.CompilerParams(use_tc_tiling_on_sc=False),
)
def _sc_embed(goods_ids, shop_ids, cate_ids, price_ids,
              goods_table, shop_table, cate_table, price_table,
              out, idx_v, rows_v, sem):
    wid = lax.axis_index("s") * NC + lax.axis_index("c")
    ids = (goods_ids, shop_ids, cate_ids, price_ids)
    tables = (goods_table, shop_table, cate_table, price_table)

    def super_body(sc):
        row_base = pl.multiple_of(wid * PER_W + sc * SUPER, SUPER)
        idrow_base = pl.multiple_of(row_base // 128, IDROWS)
        for f in range(NF):
            pltpu.sync_copy(ids[f].at[pl.ds(idrow_base, IDROWS)],
                            idx_v.at[f])
        for f in range(NF):
            copies = []
            for j in range(IDROWS):
                copies.append(pltpu.async_copy(
                    tables[f].at[idx_v.at[f, j]],
                    rows_v.at[pl.ds(j * 128, 128)],
                    sem,
                ))
            for cp in copies:
                cp.wait()
            pltpu.sync_copy(rows_v,
                            out.at[pl.ds(row_base, SUPER), pl.ds(f * D, D)])

    pl.loop(0, N_SUPER)(super_body)


def kernel(goods_ids, shop_ids, cate_ids, goods_prices,
           goods_table, shop_table, cate_table, price_table):
    ids2d = [x.reshape(N // 128, 128) for x in
             (goods_ids, shop_ids, cate_ids, goods_prices)]
    out = _sc_embed(*ids2d, goods_table, shop_table, cate_table, price_table)
    return out.reshape(B, L, NF * D)


# speedup vs baseline: 11.7322x; 1.2440x over previous
import functools

import jax
import jax.numpy as jnp
from jax import lax
from jax.experimental import pallas as pl
from jax.experimental.pallas import tpu as pltpu
from jax.experimental.pallas import tpu_sc as plsc

B, L, D = 4096, 200, 32
N = B * L
NF = 4
SMALL_V = 1000

NC, NS = 2, 16
NW = NC * NS
PER_W = N // NW

IDROWS = 8
SUPER = IDROWS * 128
N_SUPER = PER_W // SUPER

_mesh = plsc.VectorSubcoreMesh(core_axis_name="c", subcore_axis_name="s")


@functools.partial(
    pl.kernel,
    mesh=_mesh,
    out_type=jax.ShapeDtypeStruct((N, NF * D), jnp.float32),
    scratch_types=[
        pltpu.VMEM((NF, IDROWS, 128), jnp.int32),
        pltpu.VMEM((2, SUPER, D), jnp.float32),
        pltpu.VMEM_SHARED((2, SMALL_V, D), jnp.float32),
        pltpu.SemaphoreType.DMA,
        pltpu.SemaphoreType.DMA,
        pltpu.SemaphoreType.DMA,
    ],
    compiler_params=pltpu.CompilerParams(use_tc_tiling_on_sc=False),
)
def _sc_embed(goods_ids, shop_ids, cate_ids, price_ids,
              goods_table, shop_table, cate_table, price_table,
              out, idx_v, rows_v, small_sh, gsem, ssem0, ssem1):
    sid = lax.axis_index("s")
    wid = sid * NC + lax.axis_index("c")
    ids = (goods_ids, shop_ids, cate_ids, price_ids)
    ssems = (ssem0, ssem1)

    @pl.when(sid == 0)
    def _():
        pltpu.sync_copy(cate_table, small_sh.at[0])
        pltpu.sync_copy(price_table, small_sh.at[1])
    plsc.subcore_barrier()

    tables = (goods_table, shop_table, small_sh.at[0], small_sh.at[1])

    def super_body(sc):
        row_base = pl.multiple_of(wid * PER_W + sc * SUPER, SUPER)
        idrow_base = pl.multiple_of(row_base // 128, IDROWS)
        for f in range(NF):
            pltpu.sync_copy(ids[f].at[pl.ds(idrow_base, IDROWS)],
                            idx_v.at[f])
        for f in range(NF):
            cur = f % 2
            dst = out.at[pl.ds(row_base, SUPER), pl.ds(f * D, D)]
            drain = pltpu.make_async_copy(rows_v.at[cur], dst, ssems[cur])
            if f < 2:
                @pl.when(sc != 0)
                def _():
                    drain.wait()
            else:
                drain.wait()
            gathers = [
                pltpu.async_copy(
                    tables[f].at[idx_v.at[f, j]],
                    rows_v.at[cur, pl.ds(j * 128, 128)],
                    gsem,
                )
                for j in range(IDROWS)
            ]
            for cp in gathers:
                cp.wait()
            pltpu.async_copy(rows_v.at[cur], dst, ssems[cur])

    pl.loop(0, N_SUPER)(super_body)

    tail = out.at[pl.ds(0, SUPER), pl.ds(0, D)]
    pltpu.make_async_copy(rows_v.at[0], tail, ssem0).wait()
    pltpu.make_async_copy(rows_v.at[1], tail, ssem1).wait()


def kernel(goods_ids, shop_ids, cate_ids, goods_prices,
           goods_table, shop_table, cate_table, price_table):
    ids2d = [x.reshape(N // 128, 128) for x in
             (goods_ids, shop_ids, cate_ids, goods_prices)]
    out = _sc_embed(*ids2d, goods_table, shop_table, cate_table, price_table)
    return out.reshape(B, L, NF * D)


# speedup vs baseline: 13.5498x; 1.1549x over previous
import functools

import jax
import jax.numpy as jnp
from jax import lax
from jax.experimental import pallas as pl
from jax.experimental.pallas import tpu as pltpu
from jax.experimental.pallas import tpu_sc as plsc

B, L, D = 4096, 200, 32
N = B * L
NF = 4
SMALL_V = 1000

NC, NS = 2, 16
NW = NC * NS
PER_W = N // NW

IDROWS = 8
SUPER = IDROWS * 128
N_SUPER = PER_W // SUPER

_mesh = plsc.VectorSubcoreMesh(core_axis_name="c", subcore_axis_name="s")
_params = pltpu.CompilerParams(use_tc_tiling_on_sc=False)


def _field_loop(wid, fields, out, idx_v, rows_v, gsem, ssem):
    nf = len(fields)

    def super_body(sc):
        row_base = pl.multiple_of(wid * PER_W + sc * SUPER, SUPER)
        idrow_base = pl.multiple_of(row_base // 128, IDROWS)
        for f, (ids_hbm, _, _) in enumerate(fields):
            pltpu.sync_copy(ids_hbm.at[pl.ds(idrow_base, IDROWS)],
                            idx_v.at[f])
        for f, (_, table, col) in enumerate(fields):
            step = sc * nf + f
            cur = lax.rem(step, 2)
            dst = out.at[pl.ds(row_base, SUPER), pl.ds(col, D)]
            drain = pltpu.make_async_copy(rows_v.at[0], dst, ssem)
            if f >= 2:
                drain.wait()
            else:
                @pl.when(step >= 2)
                def _():
                    drain.wait()
            gathers = [
                pltpu.async_copy(
                    table.at[idx_v.at[f, j]],
                    rows_v.at[cur, pl.ds(j * 128, 128)],
                    gsem,
                )
                for j in range(IDROWS)
            ]
            for cp in gathers:
                cp.wait()
            pltpu.async_copy(rows_v.at[cur], dst, ssem)

    pl.loop(0, N_SUPER)(super_body)

    tail = out.at[pl.ds(0, SUPER), pl.ds(0, D)]
    pltpu.make_async_copy(rows_v.at[0], tail, ssem).wait()
    pltpu.make_async_copy(rows_v.at[1], tail, ssem).wait()


@functools.partial(
    pl.kernel,
    mesh=_mesh,
    out_type=jax.ShapeDtypeStruct((N, NF * D), jnp.float32),
    scratch_types=[
        pltpu.VMEM((3, IDROWS, 128), jnp.int32),
        pltpu.VMEM((2, SUPER, D), jnp.float32),
        pltpu.VMEM_SHARED((2, SMALL_V, D), jnp.float32),
        pltpu.SemaphoreType.DMA,
        pltpu.SemaphoreType.DMA,
    ],
    compiler_params=_params,
)
def _sc_embed_scp(shop_ids, cate_ids, price_ids,
                  shop_table, cate_table, price_table,
                  out, idx_v, rows_v, small_sh, gsem, ssem):
    sid = lax.axis_index("s")
    wid = sid * NC + lax.axis_index("c")

    @pl.when(sid == 0)
    def _():
        pltpu.sync_copy(cate_table, small_sh.at[0])
        pltpu.sync_copy(price_table, small_sh.at[1])
    plsc.subcore_barrier()

    fields = (
        (shop_ids, shop_table, 1 * D),
        (cate_ids, small_sh.at[0], 2 * D),
        (price_ids, small_sh.at[1], 3 * D),
    )
    _field_loop(wid, fields, out, idx_v, rows_v, gsem, ssem)


@functools.partial(
    pl.kernel,
    mesh=_mesh,
    scratch_types=[
        pltpu.VMEM((1, IDROWS, 128), jnp.int32),
        pltpu.VMEM((2, SUPER, D), jnp.float32),
        pltpu.SemaphoreType.DMA,
        pltpu.SemaphoreType.DMA,
    ],
    compiler_params=_params,
)
def _sc_embed_goods(goods_ids, goods_table, out,
                    idx_v, rows_v, gsem, ssem):
    wid = lax.axis_index("s") * NC + lax.axis_index("c")
    fields = ((goods_ids, goods_table, 0),)
    _field_loop(wid, fields, out, idx_v, rows_v, gsem, ssem)


def kernel(goods_ids, shop_ids, cate_ids, goods_prices,
           goods_table, shop_table, cate_table, price_table):
    g2, s2, c2, p2 = (x.reshape(N // 128, 128) for x in
                      (goods_ids, shop_ids, cate_ids, goods_prices))
    out_a = _sc_embed_scp(s2, c2, p2, shop_table, cate_table, price_table)
    ref = jax.new_ref(out_a)
    _sc_embed_goods(g2, goods_table, ref)
    return ref[...].reshape(B, L, NF * D)


# speedup vs baseline: 13.5718x; 1.0016x over previous
import functools

import jax
import jax.numpy as jnp
from jax import lax
from jax.experimental import pallas as pl
from jax.experimental.pallas import tpu as pltpu
from jax.experimental.pallas import tpu_sc as plsc

B, L, D = 4096, 200, 32
N = B * L
NF = 4
SMALL_V = 1000

NC, NS = 2, 16
NW = NC * NS
PER_W = N // NW

IDROWS = 8
SUPER = IDROWS * 128
N_SUPER = PER_W // SUPER

_mesh = plsc.VectorSubcoreMesh(core_axis_name="c", subcore_axis_name="s")
_params = pltpu.CompilerParams(use_tc_tiling_on_sc=False)


def _field_loop(wid, fields, out, idx_v, rows_v, gsem, ssems):
    nf = len(fields)

    def one_super(sc, u, first_body):
        row_base = pl.multiple_of(wid * PER_W + sc * SUPER, SUPER)
        idrow_base = pl.multiple_of(row_base // 128, IDROWS)
        for f, (ids_hbm, _, _) in enumerate(fields):
            pltpu.sync_copy(ids_hbm.at[pl.ds(idrow_base, IDROWS)],
                            idx_v.at[f])
        for f, (_, table, col) in enumerate(fields):
            par = (u * nf + f) % 2
            dst = out.at[pl.ds(row_base, SUPER), pl.ds(col, D)]
            drain = pltpu.make_async_copy(rows_v.at[par], dst, ssems[par])
            if first_body is None or u * nf + f >= 2:
                drain.wait()
            else:
                @pl.when(jnp.logical_not(first_body))
                def _():
                    drain.wait()
            gathers = [
                pltpu.async_copy(
                    table.at[idx_v.at[f, j]],
                    rows_v.at[par, pl.ds(j * 128, 128)],
                    gsem,
                )
                for j in range(IDROWS)
            ]
            for cp in gathers:
                cp.wait()
            pltpu.async_copy(rows_v.at[par], dst, ssems[par])

    def pair_body(p):
        for u in range(2):
            one_super(2 * p + u, u, p == 0)

    pl.loop(0, N_SUPER // 2)(pair_body)
    if N_SUPER % 2:
        one_super(N_SUPER - 1, 0, None)

    tail = out.at[pl.ds(0, SUPER), pl.ds(0, D)]
    pltpu.make_async_copy(rows_v.at[0], tail, ssems[0]).wait()
    pltpu.make_async_copy(rows_v.at[1], tail, ssems[1]).wait()


@functools.partial(
    pl.kernel,
    mesh=_mesh,
    out_type=jax.ShapeDtypeStruct((N, NF * D), jnp.float32),
    scratch_types=[
        pltpu.VMEM((3, IDROWS, 128), jnp.int32),
        pltpu.VMEM((2, SUPER, D), jnp.float32),
        pltpu.VMEM_SHARED((2, SMALL_V, D), jnp.float32),
        pltpu.SemaphoreType.DMA,
        pltpu.SemaphoreType.DMA,
        pltpu.SemaphoreType.DMA,
    ],
    compiler_params=_params,
)
def _sc_embed_scp(shop_ids, cate_ids, price_ids,
                  shop_table, cate_table, price_table,
                  out, idx_v, rows_v, small_sh, gsem, ssem0, ssem1):
    sid = lax.axis_index("s")
    wid = sid * NC + lax.axis_index("c")

    @pl.when(sid == 0)
    def _():
        pltpu.sync_copy(cate_table, small_sh.at[0])
        pltpu.sync_copy(price_table, small_sh.at[1])
    plsc.subcore_barrier()

    fields = (
        (shop_ids, shop_table, 1 * D),
        (cate_ids, small_sh.at[0], 2 * D),
        (price_ids, small_sh.at[1], 3 * D),
    )
    _field_loop(wid, fields, out, idx_v, rows_v, gsem, (ssem0, ssem1))


@functools.partial(
    pl.kernel,
    mesh=_mesh,
    scratch_types=[
        pltpu.VMEM((1, IDROWS, 128), jnp.int32),
        pltpu.VMEM((2, SUPER, D), jnp.float32),
        pltpu.SemaphoreType.DMA,
        pltpu.SemaphoreType.DMA,
        pltpu.SemaphoreType.DMA,
    ],
    compiler_params=_params,
)
def _sc_embed_goods(goods_ids, goods_table, out,
                    idx_v, rows_v, gsem, ssem0, ssem1):
    wid = lax.axis_index("s") * NC + lax.axis_index("c")
    fields = ((goods_ids, goods_table, 0),)
    _field_loop(wid, fields, out, idx_v, rows_v, gsem, (ssem0, ssem1))


def kernel(goods_ids, shop_ids, cate_ids, goods_prices,
           goods_table, shop_table, cate_table, price_table):
    g2, s2, c2, p2 = (x.reshape(N // 128, 128) for x in
                      (goods_ids, shop_ids, cate_ids, goods_prices))
    out_a = _sc_embed_scp(s2, c2, p2, shop_table, cate_table, price_table)
    ref = jax.new_ref(out_a)
    _sc_embed_goods(g2, goods_table, ref)
    return ref[...].reshape(B, L, NF * D)
